# Initial kernel scaffold; baseline (speedup 1.0000x reference)
import jax
import jax.numpy as jnp
from jax.experimental import pallas as pl


def kernel(x, emb):
    raise NotImplementedError("write your pallas kernel here")


# speedup vs baseline: 5.7517x; 5.7517x over previous
import functools

import jax
import jax.numpy as jnp
from jax import lax
from jax.experimental import pallas as pl
from jax.experimental.pallas import tpu as pltpu
from jax.experimental.pallas import tpu_sc as plsc

B, T = 16384, 200
V, D = 10, 8
N = B * T
NW = 32
N_PER_W = N // NW
CHUNK = 12800
NCHUNK = N_PER_W // CHUNK


def _sc_embed(x_hbm, emb_hbm, out_hbm, x_v, out_v, emb_v):
    wid = lax.axis_index("s") * 2 + lax.axis_index("c")
    base = wid * N_PER_W
    pltpu.sync_copy(emb_hbm, emb_v)
    iota = lax.iota(jnp.int32, 16)
    iota8 = iota * 8

    def chunk_body(ci, _):
        off = base + ci * CHUNK
        pltpu.sync_copy(x_hbm.at[pl.ds(off, CHUNK)], x_v)

        def grp(j, _):
            xv = x_v[pl.ds(j * 16, 16)]
            x8 = xv * 8
            sbase = iota8 + j * 128
            for d in range(D):
                vals = plsc.load_gather(emb_v, [x8 + d])
                plsc.store_scatter(out_v, [sbase + d], vals)
            return 0

        lax.fori_loop(0, CHUNK // 16, grp, 0)
        pltpu.sync_copy(out_v, out_hbm.at[pl.ds(off * D, CHUNK * D)])
        return 0

    lax.fori_loop(0, NCHUNK, chunk_body, 0)


@functools.partial(jax.jit, static_argnums=())
def kernel(x, emb):
    xf = x.reshape(-1).astype(jnp.int32)
    ef = emb.reshape(-1)
    mesh = plsc.VectorSubcoreMesh(core_axis_name="c", subcore_axis_name="s")
    run = pl.kernel(
        _sc_embed,
        out_type=jax.ShapeDtypeStruct((N * D,), jnp.float32),
        mesh=mesh,
        compiler_params=pltpu.CompilerParams(needs_layout_passes=False),
        scratch_types=[
            pltpu.VMEM((CHUNK,), jnp.int32),
            pltpu.VMEM((CHUNK * D,), jnp.float32),
            pltpu.VMEM((V * D,), jnp.float32),
        ],
    )
    out = run(xf, ef)
    return out.reshape(B, T, D)


# speedup vs baseline: 5.8364x; 1.0147x over previous
import functools

import jax
import jax.numpy as jnp
from jax import lax
from jax.experimental import pallas as pl
from jax.experimental.pallas import tpu as pltpu
from jax.experimental.pallas import tpu_sc as plsc

B, T = 16384, 200
V, D = 10, 8
N = B * T
NW = 32
N_PER_W = N // NW
CHUNK = 6400
NCHUNK = N_PER_W // CHUNK
NBUF = 2


def _sc_embed(x_hbm, emb_hbm, out_hbm, x_v0, x_v1, o_v0, o_v1, emb_v,
              si0, si1, so0, so1):
    wid = lax.axis_index("s") * 2 + lax.axis_index("c")
    base = wid * N_PER_W
    pltpu.sync_copy(emb_hbm, emb_v)
    iota = lax.iota(jnp.int32, 16)
    iota8 = iota * 8
    xb = (x_v0, x_v1)
    ob = (o_v0, o_v1)
    si = (si0, si1)
    so = (so0, so1)

    def in_copy(ci, b):
        return pltpu.make_async_copy(
            x_hbm.at[pl.ds(base + ci * CHUNK, CHUNK)], xb[b], si[b])

    def out_copy(ci, b):
        return pltpu.make_async_copy(
            ob[b], out_hbm.at[pl.ds((base + ci * CHUNK) * D, CHUNK * D)],
            so[b])

    in_copy(0, 0).start()
    in_copy(1, 1).start()

    def pair(ci2, _):
        for b in range(NBUF):
            ci = ci2 * NBUF + b
            in_copy(ci, b).wait()

            @pl.when(ci >= NBUF)
            def _wait_out():
                out_copy(ci - NBUF, b).wait()

            x_v = xb[b]
            o_v = ob[b]

            def _grp(jg, _):
                for u in range(4):
                    j = jg * 4 + u
                    xv = x_v[pl.ds(j * 16, 16)]
                    x8 = xv * 8
                    sb = iota8 + j * 128
                    for d in range(D):
                        vals = plsc.load_gather(emb_v, [x8 + d])
                        plsc.store_scatter(o_v, [sb + d], vals)
                return 0

            lax.fori_loop(0, CHUNK // 64, _grp, 0)

            out_copy(ci, b).start()

            @pl.when(ci + NBUF < NCHUNK)
            def _next_in():
                in_copy(ci + NBUF, b).start()
        return 0

    lax.fori_loop(0, NCHUNK // NBUF, pair, 0)
    out_copy(NCHUNK - 2, 0).wait()
    out_copy(NCHUNK - 1, 1).wait()


def kernel(x, emb):
    xf = x.reshape(-1).astype(jnp.int32)
    ef = emb.reshape(-1)
    mesh = plsc.VectorSubcoreMesh(core_axis_name="c", subcore_axis_name="s")
    run = pl.kernel(
        _sc_embed,
        out_type=jax.ShapeDtypeStruct((N * D,), jnp.float32),
        mesh=mesh,
        compiler_params=pltpu.CompilerParams(needs_layout_passes=False),
        scratch_types=[
            pltpu.VMEM((CHUNK,), jnp.int32),
            pltpu.VMEM((CHUNK,), jnp.int32),
            pltpu.VMEM((CHUNK * D,), jnp.float32),
            pltpu.VMEM((CHUNK * D,), jnp.float32),
            pltpu.VMEM((V * D,), jnp.float32),
            pltpu.SemaphoreType.DMA,
            pltpu.SemaphoreType.DMA,
            pltpu.SemaphoreType.DMA,
            pltpu.SemaphoreType.DMA,
        ],
    )
    out = run(xf, ef)
    return out.reshape(B, T, D)


# speedup vs baseline: 6.5264x; 1.1182x over previous
import functools

import jax
import jax.numpy as jnp
from jax import lax
from jax.experimental import pallas as pl
from jax.experimental.pallas import tpu as pltpu
from jax.experimental.pallas import tpu_sc as plsc

B, T = 16384, 200
V, D = 10, 8
N = B * T
NW = 32
N_PER_W = N // NW
CHUNK = 6400
NCHUNK = N_PER_W // CHUNK
NBUF = 2


def _sc_embed(x_hbm, emb_hbm, out_hbm, x_v0, x_v1, o_v0, o_v1, emb_v,
              si0, si1, so0, so1):
    wid = lax.axis_index("s") * 2 + lax.axis_index("c")
    base = wid * N_PER_W
    pltpu.sync_copy(emb_hbm, emb_v)
    iota = lax.iota(jnp.int32, 16)
    iota8 = iota * 8
    xb = (x_v0, x_v1)
    ob = (o_v0, o_v1)
    si = (si0, si1)
    so = (so0, so1)

    def in_copy(ci, b):
        return pltpu.make_async_copy(
            x_hbm.at[pl.ds(base + ci * CHUNK, CHUNK)], xb[b], si[b])

    def out_copy(ci, b):
        return pltpu.make_async_copy(
            ob[b], out_hbm.at[pl.ds((base + ci * CHUNK) * D, CHUNK * D)],
            so[b])

    in_copy(0, 0).start()
    in_copy(1, 1).start()

    def pair(ci2, _):
        for b in range(NBUF):
            ci = ci2 * NBUF + b
            in_copy(ci, b).wait()

            @pl.when(ci >= NBUF)
            def _wait_out():
                out_copy(ci - NBUF, b).wait()

            x_v = xb[b]
            o_v = ob[b]

            @plsc.parallel_loop(0, CHUNK // 16, unroll=8)
            def _grp(j):
                xv = x_v[pl.ds(j * 16, 16)]
                x8 = xv * 8
                sb = iota8 + j * 128
                for d in range(D):
                    vals = plsc.load_gather(emb_v, [x8 + d])
                    plsc.store_scatter(o_v, [sb + d], vals)

            out_copy(ci, b).start()

            @pl.when(ci + NBUF < NCHUNK)
            def _next_in():
                in_copy(ci + NBUF, b).start()
        return 0

    lax.fori_loop(0, NCHUNK // NBUF, pair, 0)
    out_copy(NCHUNK - 2, 0).wait()
    out_copy(NCHUNK - 1, 1).wait()


def kernel(x, emb):
    xf = x.reshape(-1).astype(jnp.int32)
    ef = emb.reshape(-1)
    mesh = plsc.VectorSubcoreMesh(core_axis_name="c", subcore_axis_name="s")
    run = pl.kernel(
        _sc_embed,
        out_type=jax.ShapeDtypeStruct((N * D,), jnp.float32),
        mesh=mesh,
        compiler_params=pltpu.CompilerParams(needs_layout_passes=False),
        scratch_types=[
            pltpu.VMEM((CHUNK,), jnp.int32),
            pltpu.VMEM((CHUNK,), jnp.int32),
            pltpu.VMEM((CHUNK * D,), jnp.float32),
            pltpu.VMEM((CHUNK * D,), jnp.float32),
            pltpu.VMEM((V * D,), jnp.float32),
            pltpu.SemaphoreType.DMA,
            pltpu.SemaphoreType.DMA,
            pltpu.SemaphoreType.DMA,
            pltpu.SemaphoreType.DMA,
        ],
    )
    out = run(xf, ef)
    return out.reshape(B, T, D)


# speedup vs baseline: 68.9614x; 10.5665x over previous
import jax
import jax.numpy as jnp
from jax import lax
from jax.experimental import pallas as pl
from jax.experimental.pallas import tpu as pltpu
from jax.experimental.pallas import tpu_sc as plsc

B, T = 16384, 200
V, D = 10, 8
N = B * T
NW = 32
N_PER_W = N // NW
CHUNK = 6400
NCHUNK = N_PER_W // CHUNK
NBUF = 2


def _sc_embed(x_hbm, emb_hbm, out_hbm, x_v0, x_v1, o_v0, o_v1, emb_v,
              si0, si1, so0, so1):
    wid = lax.axis_index("s") * 2 + lax.axis_index("c")
    base = wid * N_PER_W
    pltpu.sync_copy(emb_hbm, emb_v)
    iota = lax.iota(jnp.int32, 16)
    xb = (x_v0, x_v1)
    ob = (o_v0, o_v1)
    si = (si0, si1)
    so = (so0, so1)

    def in_copy(ci, b):
        return pltpu.make_async_copy(
            x_hbm.at[pl.ds(base + ci * CHUNK, CHUNK)], xb[b], si[b])

    def out_copy(ci, b):
        return pltpu.make_async_copy(
            ob[b], out_hbm.at[pl.ds((base + ci * CHUNK) * D, CHUNK * D)],
            so[b])

    in_copy(0, 0).start()
    in_copy(1, 1).start()

    def pair(ci2, _):
        for b in range(NBUF):
            ci = ci2 * NBUF + b
            in_copy(ci, b).wait()

            @pl.when(ci >= NBUF)
            def _wait_out():
                out_copy(ci - NBUF, b).wait()

            x_v = xb[b]
            o_v = ob[b]

            @plsc.parallel_loop(0, CHUNK // 16, unroll=8)
            def _grp(q):
                xv = x_v[pl.ds(q * 16, 16)]
                x8 = xv * 8
                sb = iota + ((q >> 3) << 10) + ((q & 7) << 4)
                for d in range(D):
                    vals = plsc.load_gather(emb_v, [x8 + d])
                    plsc.store_scatter(o_v, [sb + d * 128], vals)

            out_copy(ci, b).start()

            @pl.when(ci + NBUF < NCHUNK)
            def _next_in():
                in_copy(ci + NBUF, b).start()
        return 0

    lax.fori_loop(0, NCHUNK // NBUF, pair, 0)
    out_copy(NCHUNK - 2, 0).wait()
    out_copy(NCHUNK - 1, 1).wait()


def kernel(x, emb):
    xtf = jnp.swapaxes(x, 0, 1).reshape(-1).astype(jnp.int32)
    ef = emb.reshape(-1)
    mesh = plsc.VectorSubcoreMesh(core_axis_name="c", subcore_axis_name="s")
    run = pl.kernel(
        _sc_embed,
        out_type=jax.ShapeDtypeStruct((N * D,), jnp.float32),
        mesh=mesh,
        compiler_params=pltpu.CompilerParams(needs_layout_passes=False),
        scratch_types=[
            pltpu.VMEM((CHUNK,), jnp.int32),
            pltpu.VMEM((CHUNK,), jnp.int32),
            pltpu.VMEM((CHUNK * D,), jnp.float32),
            pltpu.VMEM((CHUNK * D,), jnp.float32),
            pltpu.VMEM((V * D,), jnp.float32),
            pltpu.SemaphoreType.DMA,
            pltpu.SemaphoreType.DMA,
            pltpu.SemaphoreType.DMA,
            pltpu.SemaphoreType.DMA,
        ],
    )
    out = run(xtf, ef)
    return (
        out.reshape(T, B // 128, D, 128)
        .transpose(1, 3, 0, 2)
        .reshape(B, T, D)
    )


# speedup vs baseline: 78.0314x; 1.1315x over previous
import jax
import jax.numpy as jnp
from jax import lax
from jax.experimental import pallas as pl
from jax.experimental.pallas import tpu as pltpu
from jax.experimental.pallas import tpu_sc as plsc

B, T = 16384, 200
V, D = 10, 8
N = B * T
NW = 32
N_PER_W = N // NW
CHUNK = 6400
NCHUNK = N_PER_W // CHUNK
NBUF = 2


def _sc_embed(x_hbm, emb_hbm, out_hbm, x_v0, x_v1, o_v0, o_v1, emb_v,
              si0, si1, so0, so1):
    wid = lax.axis_index("s") * 2 + lax.axis_index("c")
    base = wid * N_PER_W
    pltpu.sync_copy(emb_hbm, emb_v)
    xb = (x_v0, x_v1)
    ob = (o_v0, o_v1)
    si = (si0, si1)
    so = (so0, so1)

    def in_copy(ci, b):
        return pltpu.make_async_copy(
            x_hbm.at[pl.ds(base + ci * CHUNK, CHUNK)], xb[b], si[b])

    def out_copy(ci, b):
        return pltpu.make_async_copy(
            ob[b], out_hbm.at[pl.ds((base + ci * CHUNK) * D, CHUNK * D)],
            so[b])

    in_copy(0, 0).start()
    in_copy(1, 1).start()

    def pair(ci2, _):
        for b in range(NBUF):
            ci = ci2 * NBUF + b
            in_copy(ci, b).wait()

            @pl.when(ci >= NBUF)
            def _wait_out():
                out_copy(ci - NBUF, b).wait()

            x_v = xb[b]
            o_v = ob[b]

            @plsc.parallel_loop(0, CHUNK // 16, unroll=8)
            def _grp(q):
                xv = x_v[pl.ds(q * 16, 16)]
                x8 = xv * 8
                off = ((q >> 3) << 10) + ((q & 7) << 4)
                for d in range(D):
                    vals = plsc.load_gather(emb_v, [x8 + d])
                    o_v[pl.ds(off + d * 128, 16)] = vals

            out_copy(ci, b).start()

            @pl.when(ci + NBUF < NCHUNK)
            def _next_in():
                in_copy(ci + NBUF, b).start()
        return 0

    lax.fori_loop(0, NCHUNK // NBUF, pair, 0)
    out_copy(NCHUNK - 2, 0).wait()
    out_copy(NCHUNK - 1, 1).wait()


def kernel(x, emb):
    xtf = jnp.swapaxes(x, 0, 1).reshape(-1).astype(jnp.int32)
    ef = emb.reshape(-1)
    mesh = plsc.VectorSubcoreMesh(core_axis_name="c", subcore_axis_name="s")
    run = pl.kernel(
        _sc_embed,
        out_type=jax.ShapeDtypeStruct((N * D,), jnp.float32),
        mesh=mesh,
        compiler_params=pltpu.CompilerParams(needs_layout_passes=False),
        scratch_types=[
            pltpu.VMEM((CHUNK,), jnp.int32),
            pltpu.VMEM((CHUNK,), jnp.int32),
            pltpu.VMEM((CHUNK * D,), jnp.float32),
            pltpu.VMEM((CHUNK * D,), jnp.float32),
            pltpu.VMEM((V * D,), jnp.float32),
            pltpu.SemaphoreType.DMA,
            pltpu.SemaphoreType.DMA,
            pltpu.SemaphoreType.DMA,
            pltpu.SemaphoreType.DMA,
        ],
    )
    out = run(xtf, ef)
    return (
        out.reshape(T, B // 128, D, 128)
        .transpose(1, 3, 0, 2)
        .reshape(B, T, D)
    )


# speedup vs baseline: 160.2515x; 2.0537x over previous
import jax
import jax.numpy as jnp
from jax import lax
from jax.experimental import pallas as pl
from jax.experimental.pallas import tpu as pltpu
from jax.experimental.pallas import tpu_sc as plsc

B, T = 16384, 200
V, D = 10, 8
N = B * T
NW = 32
N_PER_W = N // NW
CHUNK = 6400
NCHUNK = N_PER_W // CHUNK
NBUF = 2


def _sc_embed(x_hbm, emb_hbm, out_hbm, x_v0, x_v1, o_v0, o_v1, emb_v,
              si0, si1, so0, so1):
    wid = lax.axis_index("s") * 2 + lax.axis_index("c")
    base = wid * N_PER_W
    pltpu.sync_copy(emb_hbm, emb_v)
    iota = lax.iota(jnp.int32, 16)
    xb = (x_v0, x_v1)
    ob = (o_v0, o_v1)
    si = (si0, si1)
    so = (so0, so1)

    def in_copy(ci, b):
        return pltpu.make_async_copy(
            x_hbm.at[pl.ds(base + ci * CHUNK, CHUNK)], xb[b], si[b])

    def out_copy(ci, b):
        return pltpu.make_async_copy(
            ob[b], out_hbm.at[pl.ds((base + ci * CHUNK) * D, CHUNK * D)],
            so[b])

    in_copy(0, 0).start()
    in_copy(1, 1).start()

    def pair(ci2, _):
        for b in range(NBUF):
            ci = ci2 * NBUF + b
            in_copy(ci, b).wait()

            @pl.when(ci >= NBUF)
            def _wait_out():
                out_copy(ci - NBUF, b).wait()

            x_v = xb[b]
            o_v = ob[b]

            @plsc.parallel_loop(0, CHUNK // 16, unroll=8)
            def _grp(q):
                xv = x_v[pl.ds(q * 16, 16)]
                x128 = xv * 128
                off = ((q >> 3) << 10) + ((q & 7) << 4)
                for d in range(D):
                    vals = plsc.load_gather(emb_v, [x128 + (iota + d * 16)])
                    o_v[pl.ds(off + d * 128, 16)] = vals

            out_copy(ci, b).start()

            @pl.when(ci + NBUF < NCHUNK)
            def _next_in():
                in_copy(ci + NBUF, b).start()
        return 0

    lax.fori_loop(0, NCHUNK // NBUF, pair, 0)
    out_copy(NCHUNK - 2, 0).wait()
    out_copy(NCHUNK - 1, 1).wait()


def kernel(x, emb):
    xtf = jnp.swapaxes(x, 0, 1).reshape(-1).astype(jnp.int32)
    ef = jnp.tile(emb.reshape(-1, 1), (1, 16)).reshape(-1)
    mesh = plsc.VectorSubcoreMesh(core_axis_name="c", subcore_axis_name="s")
    run = pl.kernel(
        _sc_embed,
        out_type=jax.ShapeDtypeStruct((N * D,), jnp.float32),
        mesh=mesh,
        compiler_params=pltpu.CompilerParams(needs_layout_passes=False),
        scratch_types=[
            pltpu.VMEM((CHUNK,), jnp.int32),
            pltpu.VMEM((CHUNK,), jnp.int32),
            pltpu.VMEM((CHUNK * D,), jnp.float32),
            pltpu.VMEM((CHUNK * D,), jnp.float32),
            pltpu.VMEM((V * D * 16,), jnp.float32),
            pltpu.SemaphoreType.DMA,
            pltpu.SemaphoreType.DMA,
            pltpu.SemaphoreType.DMA,
            pltpu.SemaphoreType.DMA,
        ],
    )
    out = run(xtf, ef)
    return (
        out.reshape(T, B // 128, D, 128)
        .transpose(1, 3, 0, 2)
        .reshape(B, T, D)
    )


# speedup vs baseline: 198.4281x; 1.2382x over previous
import jax
import jax.numpy as jnp
from jax import lax
from jax.experimental import pallas as pl
from jax.experimental.pallas import tpu as pltpu
from jax.experimental.pallas import tpu_sc as plsc

B, T = 16384, 200
V, D = 10, 8
N = B * T
NW = 32
JB = B // 128 // NW
NCHUNK = T // 8
XC = JB * 1024
OC = XC * D
NBUF = 2


def _sc_embed(x_hbm, emb_hbm, out_hbm, x_v0, x_v1, o_v0, o_v1, emb_v,
              si0, si1, so0, so1):
    wid = lax.axis_index("s") * 2 + lax.axis_index("c")
    pltpu.sync_copy(emb_hbm, emb_v)
    iota = lax.iota(jnp.int32, 16)
    xb = (x_v0, x_v1)
    ob = (o_v0, o_v1)
    si = (si0, si1)
    so = (so0, so1)

    def in_copy(ci, b):
        return pltpu.make_async_copy(
            x_hbm.at[pl.ds(ci * (128 * 1024) + wid * XC, XC)], xb[b], si[b])

    def out_copies(ci, b):
        return [
            pltpu.make_async_copy(
                ob[b].at[pl.ds(tr * XC, XC)],
                out_hbm.at[pl.ds(ci * (8 * 128 * 1024) + tr * (128 * 1024)
                                 + wid * XC, XC)],
                so[b])
            for tr in range(8)
        ]

    def process(ci, b):
        in_copy(ci, b).wait()

        @pl.when(ci >= NBUF)
        def _wait_out():
            for cp in out_copies(ci - NBUF, b):
                cp.wait()

        x_v = xb[b]
        o_v = ob[b]

        @plsc.parallel_loop(0, XC // 16, unroll=8)
        def _grp(q):
            xv = x_v[pl.ds(q * 16, 16)]
            x128 = xv * 128
            off = ((q >> 3) & 7) * XC + (q >> 6) * 1024 + (q & 7) * 16
            for d in range(D):
                vals = plsc.load_gather(emb_v, [x128 + (iota + d * 16)])
                o_v[pl.ds(off + d * 128, 16)] = vals

        for cp in out_copies(ci, b):
            cp.start()

        @pl.when(ci + NBUF < NCHUNK)
        def _next_in():
            in_copy(ci + NBUF, b).start()

    in_copy(0, 0).start()
    in_copy(1, 1).start()

    def pair(ci2, _):
        for b in range(NBUF):
            process(ci2 * NBUF + b, b)
        return 0

    lax.fori_loop(0, (NCHUNK - 1) // NBUF, pair, 0)
    process(NCHUNK - 1, 0)
    for cp in out_copies(NCHUNK - 2, 1):
        cp.wait()
    for cp in out_copies(NCHUNK - 1, 0):
        cp.wait()


def kernel(x, emb):
    xr = (x.astype(jnp.int32)
          .reshape(128, 128, 25, 8)
          .transpose(2, 0, 3, 1)
          .reshape(-1))
    ef = jnp.tile(emb.reshape(-1, 1), (1, 16)).reshape(-1)
    mesh = plsc.VectorSubcoreMesh(core_axis_name="c", subcore_axis_name="s")
    run = pl.kernel(
        _sc_embed,
        out_type=jax.ShapeDtypeStruct((N * D,), jnp.float32),
        mesh=mesh,
        compiler_params=pltpu.CompilerParams(needs_layout_passes=False),
        scratch_types=[
            pltpu.VMEM((XC,), jnp.int32),
            pltpu.VMEM((XC,), jnp.int32),
            pltpu.VMEM((OC,), jnp.float32),
            pltpu.VMEM((OC,), jnp.float32),
            pltpu.VMEM((V * D * 16,), jnp.float32),
            pltpu.SemaphoreType.DMA,
            pltpu.SemaphoreType.DMA,
            pltpu.SemaphoreType.DMA,
            pltpu.SemaphoreType.DMA,
        ],
    )
    out = run(xr, ef)
    return (
        out.reshape(T, B // 128, D, 128)
        .transpose(1, 3, 0, 2)
        .reshape(B, T, D)
    )


# speedup vs baseline: 201.0733x; 1.0133x over previous
import jax
import jax.numpy as jnp
from jax import lax
from jax.experimental import pallas as pl
from jax.experimental.pallas import tpu as pltpu
from jax.experimental.pallas import tpu_sc as plsc

B, T = 16384, 200
V, D = 10, 8
N = B * T
NW = 32
JB = B // 128 // NW
NCHUNK = T // 8
XC = JB * 1024
OC = XC * D
NBUF = 2


def _sc_embed(x_hbm, emb_hbm, out_hbm, x_v0, x_v1, o_v0, o_v1, emb_v,
              si0, si1, so0, so1):
    wid = lax.axis_index("s") * 2 + lax.axis_index("c")
    pltpu.sync_copy(emb_hbm, emb_v)
    tab = [emb_v[pl.ds(d * 16, 16)] for d in range(D)]
    xb = (x_v0, x_v1)
    ob = (o_v0, o_v1)
    si = (si0, si1)
    so = (so0, so1)

    def in_copy(ci, b):
        return pltpu.make_async_copy(
            x_hbm.at[pl.ds(ci * (128 * 1024) + wid * XC, XC)], xb[b], si[b])

    def out_copies(ci, b):
        return [
            pltpu.make_async_copy(
                ob[b].at[pl.ds(tr * XC, XC)],
                out_hbm.at[pl.ds(ci * (8 * 128 * 1024) + tr * (128 * 1024)
                                 + wid * XC, XC)],
                so[b])
            for tr in range(8)
        ]

    def process(ci, b):
        in_copy(ci, b).wait()

        @pl.when(ci >= NBUF)
        def _wait_out():
            for cp in out_copies(ci - NBUF, b):
                cp.wait()

        x_v = xb[b]
        o_v = ob[b]

        @plsc.parallel_loop(0, XC // 16, unroll=8)
        def _grp(q):
            xv = x_v[pl.ds(q * 16, 16)]
            off = ((q >> 3) & 7) * XC + (q >> 6) * 1024 + (q & 7) * 16
            for d in range(D):
                vals = jnp.take_along_axis(tab[d], xv, axis=0)
                o_v[pl.ds(off + d * 128, 16)] = vals

        for cp in out_copies(ci, b):
            cp.start()

        @pl.when(ci + NBUF < NCHUNK)
        def _next_in():
            in_copy(ci + NBUF, b).start()

    in_copy(0, 0).start()
    in_copy(1, 1).start()

    def pair(ci2, _):
        for b in range(NBUF):
            process(ci2 * NBUF + b, b)
        return 0

    lax.fori_loop(0, (NCHUNK - 1) // NBUF, pair, 0)
    process(NCHUNK - 1, 0)
    for cp in out_copies(NCHUNK - 2, 1):
        cp.wait()
    for cp in out_copies(NCHUNK - 1, 0):
        cp.wait()


def kernel(x, emb):
    xr = (x.astype(jnp.int32)
          .reshape(128, 128, 25, 8)
          .transpose(2, 0, 3, 1)
          .reshape(-1))
    ef = jnp.pad(emb.T, ((0, 0), (0, 6))).reshape(-1)
    mesh = plsc.VectorSubcoreMesh(core_axis_name="c", subcore_axis_name="s")
    run = pl.kernel(
        _sc_embed,
        out_type=jax.ShapeDtypeStruct((N * D,), jnp.float32),
        mesh=mesh,
        compiler_params=pltpu.CompilerParams(needs_layout_passes=False),
        scratch_types=[
            pltpu.VMEM((XC,), jnp.int32),
            pltpu.VMEM((XC,), jnp.int32),
            pltpu.VMEM((OC,), jnp.float32),
            pltpu.VMEM((OC,), jnp.float32),
            pltpu.VMEM((D * 16,), jnp.float32),
            pltpu.SemaphoreType.DMA,
            pltpu.SemaphoreType.DMA,
            pltpu.SemaphoreType.DMA,
            pltpu.SemaphoreType.DMA,
        ],
    )
    out = run(xr, ef)
    return (
        out.reshape(T, B // 128, D, 128)
        .transpose(1, 3, 0, 2)
        .reshape(B, T, D)
    )
